# baseline (speedup 1.0000x reference)
import functools

import jax
import jax.numpy as jnp
from jax import lax
from jax.experimental import pallas as pl
from jax.experimental.pallas import tpu as pltpu
from jax.experimental.pallas import tpu_sc as plsc

N = 10000
E = 320000
D = 128
H = 64
C = 40

NC = 2
NS = 16
CH = 128
TCH = E // CH
T0 = 1250
T1 = TCH - T0
Q0, R0 = T0 // NS, T0 % NS
Q1, R1 = T1 // NS, T1 % NS
NCHMAX = max(Q0, Q1) + 1
CHROWS = TCH + NCHMAX + 8
NPAD = 10240
RPT = NPAD // NS
PAD_IDX = NPAD - 1

BM = 1024
BMF = 1000


def _mesh():
    return plsc.VectorSubcoreMesh(core_axis_name="c", subcore_axis_name="s")


_SC_PARAMS = pltpu.CompilerParams(use_tc_tiling_on_sc=False)


def _tile_range(c, s):
    q = jnp.where(c == 0, Q0, Q1)
    r = jnp.where(c == 0, R0, R1)
    base = jnp.where(c == 0, 0, T0)
    start = base + s * q + jnp.minimum(s, r)
    n = q + (s < r).astype(jnp.int32)
    return start, n


def _deg_body(dstf, ones_hbm, zeros_hbm, out, dst_v, ones_v, acc):
    c = lax.axis_index("c")
    s = lax.axis_index("s")
    start, n = _tile_range(c, s)
    pltpu.sync_copy(zeros_hbm.at[pl.ds(s * RPT, RPT)], acc.at[pl.ds(s * RPT, RPT)])
    pltpu.sync_copy(dstf.at[pl.ds(start, NCHMAX)], dst_v)
    pltpu.sync_copy(ones_hbm, ones_v)
    plsc.subcore_barrier()

    def chunk(i, carry):
        pltpu.sync_copy(ones_v, acc.at[dst_v.at[i]], add=True)
        return carry

    lax.fori_loop(0, n, chunk, 0)
    plsc.subcore_barrier()
    pltpu.sync_copy(acc.at[pl.ds(s * RPT, RPT)], out.at[c, pl.ds(s * RPT, RPT)])


def _deg_call(dstf, ones16, zeros16):
    return pl.kernel(
        _deg_body,
        out_type=jax.ShapeDtypeStruct((NC, NPAD, 16), jnp.bfloat16),
        mesh=_mesh(),
        scratch_types=[
            pltpu.VMEM((NCHMAX, CH), jnp.int32),
            pltpu.VMEM((CH, 16), jnp.bfloat16),
            pltpu.VMEM_SHARED((NPAD, 16), jnp.bfloat16),
        ],
        compiler_params=_SC_PARAMS,
    )(dstf, ones16, zeros16)


def _agg_body(u_hbm, srcf, dstf, zeros_hbm, out, src_v, dst_v, rows_v, acc, sem):
    c = lax.axis_index("c")
    s = lax.axis_index("s")
    start, n = _tile_range(c, s)
    pltpu.sync_copy(zeros_hbm.at[pl.ds(s * RPT, RPT)], acc.at[pl.ds(s * RPT, RPT)])
    pltpu.sync_copy(srcf.at[pl.ds(start, NCHMAX)], src_v)
    pltpu.sync_copy(dstf.at[pl.ds(start, NCHMAX)], dst_v)
    plsc.subcore_barrier()

    def chunk(i, carry):
        pltpu.async_copy(u_hbm.at[src_v.at[i]], rows_v, sem).wait()
        pltpu.sync_copy(rows_v, acc.at[dst_v.at[i]], add=True)
        return carry

    lax.fori_loop(0, n, chunk, 0)
    plsc.subcore_barrier()
    pltpu.sync_copy(acc.at[pl.ds(s * RPT, RPT)], out.at[c, pl.ds(s * RPT, RPT)])


def _agg_call(u_bf, srcf, dstf, zeros_bf):
    return pl.kernel(
        _agg_body,
        out_type=jax.ShapeDtypeStruct((NC, NPAD, H), jnp.bfloat16),
        mesh=_mesh(),
        scratch_types=[
            pltpu.VMEM((NCHMAX, CH), jnp.int32),
            pltpu.VMEM((NCHMAX, CH), jnp.int32),
            pltpu.VMEM((CH, H), jnp.bfloat16),
            pltpu.VMEM_SHARED((NPAD, H), jnp.bfloat16),
            pltpu.SemaphoreType.DMA,
        ],
        compiler_params=_SC_PARAMS,
    )(u_bf, srcf, dstf, zeros_bf)


def _dinv_block(deg_ref, base_rows):
    deg = (deg_ref[0][:, 0:1] + deg_ref[1][:, 0:1]).astype(jnp.float32) + 1.0
    rows = base_rows + lax.broadcasted_iota(jnp.int32, deg.shape, 0)
    return jnp.where(rows < N, lax.rsqrt(deg), 0.0)


def _mm_body(x_ref, w_ref, g_ref):
    g_ref[...] = jnp.dot(x_ref[...], w_ref[...], preferred_element_type=jnp.float32)


def _mm_only(x, W1):
    return pl.pallas_call(
        _mm_body,
        grid=(NPAD // BM,),
        in_specs=[
            pl.BlockSpec((BM, D), lambda i: (i, 0)),
            pl.BlockSpec((D, H), lambda i: (0, 0)),
        ],
        out_specs=pl.BlockSpec((BM, H), lambda i: (i, 0)),
        out_shape=jax.ShapeDtypeStruct((NPAD, H), jnp.float32),
    )(x, W1)


def _scale_body(deg_ref, g_ref, ob_ref):
    i = pl.program_id(0)
    dinv = _dinv_block(deg_ref, i * BM)
    rows = i * BM + lax.broadcasted_iota(jnp.int32, (BM, 1), 0)
    u = jnp.where(rows < N, g_ref[...] * dinv, 0.0)
    ob_ref[...] = u.astype(jnp.bfloat16)


def _mm_scale(degp, g1):
    return pl.pallas_call(
        _scale_body,
        grid=(NPAD // BM,),
        in_specs=[
            pl.BlockSpec((NC, BM, 16), lambda i: (0, i, 0)),
            pl.BlockSpec((BM, H), lambda i: (i, 0)),
        ],
        out_specs=pl.BlockSpec((BM, H), lambda i: (i, 0)),
        out_shape=jax.ShapeDtypeStruct((NPAD, H), jnp.bfloat16),
    )(degp, g1)


def _comb_mm_body(deg_ref, sp_ref, u_ref, b_ref, w_ref, ob_ref):
    i = pl.program_id(0)
    dinv = _dinv_block(deg_ref, i * BM)
    t = (sp_ref[0] + sp_ref[1]).astype(jnp.float32) + u_ref[...].astype(jnp.float32)
    h = jnp.maximum(dinv * t + b_ref[...], 0.0)
    u = dinv * jnp.dot(h, w_ref[...], preferred_element_type=jnp.float32)
    ob_ref[...] = u.astype(jnp.bfloat16)


def _comb_mm(degp, sp, u, b_row, W):
    return pl.pallas_call(
        _comb_mm_body,
        grid=(NPAD // BM,),
        in_specs=[
            pl.BlockSpec((NC, BM, 16), lambda i: (0, i, 0)),
            pl.BlockSpec((NC, BM, H), lambda i: (0, i, 0)),
            pl.BlockSpec((BM, H), lambda i: (i, 0)),
            pl.BlockSpec((1, H), lambda i: (0, 0)),
            pl.BlockSpec((H, H), lambda i: (0, 0)),
        ],
        out_specs=pl.BlockSpec((BM, H), lambda i: (i, 0)),
        out_shape=jax.ShapeDtypeStruct((NPAD, H), jnp.bfloat16),
    )(degp, sp, u, b_row, W)


def _comb_body(deg_ref, sp_ref, u_ref, b_ref, ob_ref):
    i = pl.program_id(0)
    dinv = _dinv_block(deg_ref, i * BM)
    t = (sp_ref[0] + sp_ref[1]).astype(jnp.float32) + u_ref[...].astype(jnp.float32)
    u = dinv * jnp.maximum(dinv * t + b_ref[...], 0.0)
    ob_ref[...] = u.astype(jnp.bfloat16)


def _comb(degp, sp, u, b_row):
    return pl.pallas_call(
        _comb_body,
        grid=(NPAD // BM,),
        in_specs=[
            pl.BlockSpec((NC, BM, 16), lambda i: (0, i, 0)),
            pl.BlockSpec((NC, BM, H), lambda i: (0, i, 0)),
            pl.BlockSpec((BM, H), lambda i: (i, 0)),
            pl.BlockSpec((1, H), lambda i: (0, 0)),
        ],
        out_specs=pl.BlockSpec((BM, H), lambda i: (i, 0)),
        out_shape=jax.ShapeDtypeStruct((NPAD, H), jnp.bfloat16),
    )(degp, sp, u, b_row)


def _final_body(deg_ref, sp_ref, u_ref, wf_ref, bf_ref, o_ref):
    deg = (deg_ref[0][:, 0:1] + deg_ref[1][:, 0:1]).astype(jnp.float32) + 1.0
    dinv = lax.rsqrt(deg)
    t = dinv * ((sp_ref[0] + sp_ref[1]).astype(jnp.float32) + u_ref[...].astype(jnp.float32))
    z = jnp.dot(t, wf_ref[...], preferred_element_type=jnp.float32) + bf_ref[...]
    m = jnp.max(z, axis=1, keepdims=True)
    lse = m + jnp.log(jnp.sum(jnp.exp(z - m), axis=1, keepdims=True))
    o_ref[...] = z - lse


def _final(degp, sp, u, Wf, bf_row):
    return pl.pallas_call(
        _final_body,
        grid=(N // BMF,),
        in_specs=[
            pl.BlockSpec((NC, BMF, 16), lambda i: (0, i, 0)),
            pl.BlockSpec((NC, BMF, H), lambda i: (0, i, 0)),
            pl.BlockSpec((BMF, H), lambda i: (i, 0)),
            pl.BlockSpec((H, C), lambda i: (0, 0)),
            pl.BlockSpec((1, C), lambda i: (0, 0)),
        ],
        out_specs=pl.BlockSpec((BMF, C), lambda i: (i, 0)),
        out_shape=jax.ShapeDtypeStruct((N, C), jnp.float32),
    )(degp, sp, u, Wf, bf_row)


def kernel(x, edge_index, W1, b1, W2, b2, Wf, bf):
    ei = edge_index.astype(jnp.int32)
    tail = jnp.full((CHROWS * CH - E,), PAD_IDX, jnp.int32)
    srcf = jnp.concatenate([ei[0], tail]).reshape(CHROWS, CH)
    dstf = jnp.concatenate([ei[1], tail]).reshape(CHROWS, CH)

    zeros16 = jnp.zeros((NPAD, 16), jnp.bfloat16)
    zeros_bf = jnp.zeros((NPAD, H), jnp.bfloat16)
    ones16 = jnp.ones((CH, 16), jnp.bfloat16)
    b1r = b1.reshape(1, H)
    b2r = b2.reshape(1, H)
    bfr = bf.reshape(1, C)

    g1 = _mm_only(x, W1)
    degp = _deg_call(dstf, ones16, zeros16)
    u1 = _mm_scale(degp, g1)
    s1 = _agg_call(u1, srcf, dstf, zeros_bf)
    u2 = _comb_mm(degp, s1, u1, b1r, W2)
    s2 = _agg_call(u2, srcf, dstf, zeros_bf)
    u3 = _comb(degp, s2, u2, b2r)
    s3 = _agg_call(u3, srcf, dstf, zeros_bf)
    return _final(degp, s3, u3, Wf, bfr)


# speedup vs baseline: 1.0031x; 1.0031x over previous
import functools

import jax
import jax.numpy as jnp
from jax import lax
from jax.experimental import pallas as pl
from jax.experimental.pallas import tpu as pltpu
from jax.experimental.pallas import tpu_sc as plsc

N = 10000
E = 320000
D = 128
H = 64
C = 40

NC = 2
NS = 16
CH = 128
TCH = E // CH
T0 = 1250
T1 = TCH - T0
Q0, R0 = T0 // NS, T0 % NS
Q1, R1 = T1 // NS, T1 % NS
NCHMAX = max(Q0, Q1) + 1
CHROWS = TCH + NCHMAX + 8
NPAD = 10240
RPT = NPAD // NS
PAD_IDX = NPAD - 1

BM = 1024
BMF = 1000


def _mesh():
    return plsc.VectorSubcoreMesh(core_axis_name="c", subcore_axis_name="s")


_SC_PARAMS = pltpu.CompilerParams(use_tc_tiling_on_sc=False)


def _tile_range(c, s):
    q = jnp.where(c == 0, Q0, Q1)
    r = jnp.where(c == 0, R0, R1)
    base = jnp.where(c == 0, 0, T0)
    start = base + s * q + jnp.minimum(s, r)
    n = q + (s < r).astype(jnp.int32)
    return start, n


def _deg_body(dstf, ones_hbm, zeros_hbm, out, dst_v, ones_v, acc):
    c = lax.axis_index("c")
    s = lax.axis_index("s")
    start, n = _tile_range(c, s)
    pltpu.sync_copy(zeros_hbm.at[pl.ds(s * RPT, RPT)], acc.at[pl.ds(s * RPT, RPT)])
    pltpu.sync_copy(dstf.at[pl.ds(start, NCHMAX)], dst_v)
    pltpu.sync_copy(ones_hbm, ones_v)
    plsc.subcore_barrier()

    def chunk(i, carry):
        pltpu.sync_copy(ones_v, acc.at[dst_v.at[i]], add=True)
        return carry

    lax.fori_loop(0, n, chunk, 0)
    plsc.subcore_barrier()
    pltpu.sync_copy(acc.at[pl.ds(s * RPT, RPT)], out.at[c, pl.ds(s * RPT, RPT)])


def _deg_call(dstf, ones16, zeros16):
    return pl.kernel(
        _deg_body,
        out_type=jax.ShapeDtypeStruct((NC, NPAD, 16), jnp.bfloat16),
        mesh=_mesh(),
        scratch_types=[
            pltpu.VMEM((NCHMAX, CH), jnp.int32),
            pltpu.VMEM((CH, 16), jnp.bfloat16),
            pltpu.VMEM_SHARED((NPAD, 16), jnp.bfloat16),
        ],
        compiler_params=_SC_PARAMS,
    )(dstf, ones16, zeros16)


def _agg_body(u_hbm, srcf, dstf, zeros_hbm, out, src_v, dst_v, rows_v, acc, sem):
    c = lax.axis_index("c")
    s = lax.axis_index("s")
    start, n = _tile_range(c, s)
    pltpu.sync_copy(zeros_hbm.at[pl.ds(s * RPT, RPT)], acc.at[pl.ds(s * RPT, RPT)])
    pltpu.sync_copy(srcf.at[pl.ds(start, NCHMAX)], src_v)
    pltpu.sync_copy(dstf.at[pl.ds(start, NCHMAX)], dst_v)
    plsc.subcore_barrier()

    def chunk(i, carry):
        pltpu.async_copy(u_hbm.at[src_v.at[i]], rows_v, sem).wait()
        pltpu.sync_copy(rows_v, acc.at[dst_v.at[i]], add=True)
        return carry

    lax.fori_loop(0, n, chunk, 0)
    plsc.subcore_barrier()
    pltpu.sync_copy(acc.at[pl.ds(s * RPT, RPT)], out.at[c, pl.ds(s * RPT, RPT)])


def _agg_call(u_bf, srcf, dstf, zeros_bf):
    return pl.kernel(
        _agg_body,
        out_type=jax.ShapeDtypeStruct((NC, NPAD, H), jnp.bfloat16),
        mesh=_mesh(),
        scratch_types=[
            pltpu.VMEM((NCHMAX, CH), jnp.int32),
            pltpu.VMEM((NCHMAX, CH), jnp.int32),
            pltpu.VMEM((CH, H), jnp.bfloat16),
            pltpu.VMEM_SHARED((NPAD, H), jnp.bfloat16),
            pltpu.SemaphoreType.DMA,
        ],
        compiler_params=_SC_PARAMS,
    )(u_bf, srcf, dstf, zeros_bf)


def _dinv_block(deg_ref, base_rows):
    deg = (deg_ref[0][:, 0:1] + deg_ref[1][:, 0:1]).astype(jnp.float32) + 1.0
    rows = base_rows + lax.broadcasted_iota(jnp.int32, deg.shape, 0)
    return jnp.where(rows < N, lax.rsqrt(deg), 0.0)


def _mm_scale_body(deg_ref, x_ref, w_ref, ob_ref):
    i = pl.program_id(0)
    dinv = _dinv_block(deg_ref, i * BM)
    rows = i * BM + lax.broadcasted_iota(jnp.int32, (BM, 1), 0)
    g = jnp.dot(x_ref[...], w_ref[...], preferred_element_type=jnp.float32)
    u = jnp.where(rows < N, g * dinv, 0.0)
    ob_ref[...] = u.astype(jnp.bfloat16)


def _mm_scale(degp, x, W1):
    return pl.pallas_call(
        _mm_scale_body,
        grid=(NPAD // BM,),
        in_specs=[
            pl.BlockSpec((NC, BM, 16), lambda i: (0, i, 0)),
            pl.BlockSpec((BM, D), lambda i: (i, 0)),
            pl.BlockSpec((D, H), lambda i: (0, 0)),
        ],
        out_specs=pl.BlockSpec((BM, H), lambda i: (i, 0)),
        out_shape=jax.ShapeDtypeStruct((NPAD, H), jnp.bfloat16),
    )(degp, x, W1)


def _comb_mm_body(deg_ref, sp_ref, u_ref, b_ref, w_ref, ob_ref):
    i = pl.program_id(0)
    dinv = _dinv_block(deg_ref, i * BM)
    t = (sp_ref[0] + sp_ref[1]).astype(jnp.float32) + u_ref[...].astype(jnp.float32)
    h = jnp.maximum(dinv * t + b_ref[...], 0.0)
    u = dinv * jnp.dot(h, w_ref[...], preferred_element_type=jnp.float32)
    ob_ref[...] = u.astype(jnp.bfloat16)


def _comb_mm(degp, sp, u, b_row, W):
    return pl.pallas_call(
        _comb_mm_body,
        grid=(NPAD // BM,),
        in_specs=[
            pl.BlockSpec((NC, BM, 16), lambda i: (0, i, 0)),
            pl.BlockSpec((NC, BM, H), lambda i: (0, i, 0)),
            pl.BlockSpec((BM, H), lambda i: (i, 0)),
            pl.BlockSpec((1, H), lambda i: (0, 0)),
            pl.BlockSpec((H, H), lambda i: (0, 0)),
        ],
        out_specs=pl.BlockSpec((BM, H), lambda i: (i, 0)),
        out_shape=jax.ShapeDtypeStruct((NPAD, H), jnp.bfloat16),
    )(degp, sp, u, b_row, W)


def _comb_body(deg_ref, sp_ref, u_ref, b_ref, ob_ref):
    i = pl.program_id(0)
    dinv = _dinv_block(deg_ref, i * BM)
    t = (sp_ref[0] + sp_ref[1]).astype(jnp.float32) + u_ref[...].astype(jnp.float32)
    u = dinv * jnp.maximum(dinv * t + b_ref[...], 0.0)
    ob_ref[...] = u.astype(jnp.bfloat16)


def _comb(degp, sp, u, b_row):
    return pl.pallas_call(
        _comb_body,
        grid=(NPAD // BM,),
        in_specs=[
            pl.BlockSpec((NC, BM, 16), lambda i: (0, i, 0)),
            pl.BlockSpec((NC, BM, H), lambda i: (0, i, 0)),
            pl.BlockSpec((BM, H), lambda i: (i, 0)),
            pl.BlockSpec((1, H), lambda i: (0, 0)),
        ],
        out_specs=pl.BlockSpec((BM, H), lambda i: (i, 0)),
        out_shape=jax.ShapeDtypeStruct((NPAD, H), jnp.bfloat16),
    )(degp, sp, u, b_row)


def _final_body(deg_ref, sp_ref, u_ref, wf_ref, bf_ref, o_ref):
    deg = (deg_ref[0][:, 0:1] + deg_ref[1][:, 0:1]).astype(jnp.float32) + 1.0
    dinv = lax.rsqrt(deg)
    t = dinv * ((sp_ref[0] + sp_ref[1]).astype(jnp.float32) + u_ref[...].astype(jnp.float32))
    z = jnp.dot(t, wf_ref[...], preferred_element_type=jnp.float32) + bf_ref[...]
    m = jnp.max(z, axis=1, keepdims=True)
    lse = m + jnp.log(jnp.sum(jnp.exp(z - m), axis=1, keepdims=True))
    o_ref[...] = z - lse


def _final(degp, sp, u, Wf, bf_row):
    return pl.pallas_call(
        _final_body,
        grid=(N // BMF,),
        in_specs=[
            pl.BlockSpec((NC, BMF, 16), lambda i: (0, i, 0)),
            pl.BlockSpec((NC, BMF, H), lambda i: (0, i, 0)),
            pl.BlockSpec((BMF, H), lambda i: (i, 0)),
            pl.BlockSpec((H, C), lambda i: (0, 0)),
            pl.BlockSpec((1, C), lambda i: (0, 0)),
        ],
        out_specs=pl.BlockSpec((BMF, C), lambda i: (i, 0)),
        out_shape=jax.ShapeDtypeStruct((N, C), jnp.float32),
    )(degp, sp, u, Wf, bf_row)


def kernel(x, edge_index, W1, b1, W2, b2, Wf, bf):
    ei = edge_index.astype(jnp.int32)
    tail = jnp.full((CHROWS * CH - E,), PAD_IDX, jnp.int32)
    srcf = jnp.concatenate([ei[0], tail]).reshape(CHROWS, CH)
    dstf = jnp.concatenate([ei[1], tail]).reshape(CHROWS, CH)

    zeros16 = jnp.zeros((NPAD, 16), jnp.bfloat16)
    zeros_bf = jnp.zeros((NPAD, H), jnp.bfloat16)
    ones16 = jnp.ones((CH, 16), jnp.bfloat16)
    b1r = b1.reshape(1, H)
    b2r = b2.reshape(1, H)
    bfr = bf.reshape(1, C)

    degp = _deg_call(dstf, ones16, zeros16)
    u1 = _mm_scale(degp, x, W1)
    s1 = _agg_call(u1, srcf, dstf, zeros_bf)
    u2 = _comb_mm(degp, s1, u1, b1r, W2)
    s2 = _agg_call(u2, srcf, dstf, zeros_bf)
    u3 = _comb(degp, s2, u2, b2r)
    s3 = _agg_call(u3, srcf, dstf, zeros_bf)
    return _final(degp, s3, u3, Wf, bfr)


# speedup vs baseline: 1.0175x; 1.0143x over previous
import jax
import jax.numpy as jnp
from jax import lax
from jax.experimental import pallas as pl
from jax.experimental.pallas import tpu as pltpu
from jax.experimental.pallas import tpu_sc as plsc

N = 10000
E = 320000
D = 128
H = 64
C = 40

NC = 2
NS = 16
CH = 128
TCH = E // CH
T0 = 1250
T1 = TCH - T0
Q0, R0 = T0 // NS, T0 % NS
Q1, R1 = T1 // NS, T1 % NS
NCHMAX = max(Q0, Q1) + 1
CHROWS = TCH + NCHMAX + 8
NPAD = 10240
RPT = NPAD // NS
PAD_IDX = NPAD - 1

BM = 2048
BMF = 1000


def _mesh():
    return plsc.VectorSubcoreMesh(core_axis_name="c", subcore_axis_name="s")


_SC_PARAMS = pltpu.CompilerParams(use_tc_tiling_on_sc=False)


def _tile_range(c, s):
    q = jnp.where(c == 0, Q0, Q1)
    r = jnp.where(c == 0, R0, R1)
    base = jnp.where(c == 0, 0, T0)
    start = base + s * q + jnp.minimum(s, r)
    n = q + (s < r).astype(jnp.int32)
    return start, n


def _deg_body(dstf, ones_hbm, zeros_hbm, out, dst_v, ones_v, acc):
    c = lax.axis_index("c")
    s = lax.axis_index("s")
    start, n = _tile_range(c, s)
    pltpu.sync_copy(zeros_hbm.at[pl.ds(s * RPT, RPT)], acc.at[pl.ds(s * RPT, RPT)])
    pltpu.sync_copy(dstf.at[pl.ds(start, NCHMAX)], dst_v)
    pltpu.sync_copy(ones_hbm, ones_v)
    plsc.subcore_barrier()

    def chunk(i, carry):
        pltpu.sync_copy(ones_v, acc.at[dst_v.at[i]], add=True)
        return carry

    lax.fori_loop(0, n, chunk, 0)
    plsc.subcore_barrier()
    pltpu.sync_copy(acc.at[pl.ds(s * RPT, RPT)], out.at[c, pl.ds(s * RPT, RPT)])


def _deg_call(dstf, ones16, zeros16):
    return pl.kernel(
        _deg_body,
        out_type=jax.ShapeDtypeStruct((NC, NPAD, 16), jnp.bfloat16),
        mesh=_mesh(),
        scratch_types=[
            pltpu.VMEM((NCHMAX, CH), jnp.int32),
            pltpu.VMEM((CH, 16), jnp.bfloat16),
            pltpu.VMEM_SHARED((NPAD, 16), jnp.bfloat16),
        ],
        compiler_params=_SC_PARAMS,
    )(dstf, ones16, zeros16)


def _agg_body(u_hbm, srcf, dstf, zeros_hbm, out, src_v, dst_v, rows_v, acc, sem):
    c = lax.axis_index("c")
    s = lax.axis_index("s")
    start, n = _tile_range(c, s)
    pltpu.sync_copy(zeros_hbm.at[pl.ds(s * RPT, RPT)], acc.at[pl.ds(s * RPT, RPT)])
    pltpu.sync_copy(srcf.at[pl.ds(start, NCHMAX)], src_v)
    pltpu.sync_copy(dstf.at[pl.ds(start, NCHMAX)], dst_v)
    plsc.subcore_barrier()

    def chunk(i, carry):
        pltpu.async_copy(u_hbm.at[src_v.at[i]], rows_v, sem).wait()
        pltpu.sync_copy(rows_v, acc.at[dst_v.at[i]], add=True)
        return carry

    lax.fori_loop(0, n, chunk, 0)
    plsc.subcore_barrier()
    pltpu.sync_copy(acc.at[pl.ds(s * RPT, RPT)], out.at[c, pl.ds(s * RPT, RPT)])


def _agg_call(u_bf, srcf, dstf, zeros_bf):
    return pl.kernel(
        _agg_body,
        out_type=jax.ShapeDtypeStruct((NC, NPAD, H), jnp.bfloat16),
        mesh=_mesh(),
        scratch_types=[
            pltpu.VMEM((NCHMAX, CH), jnp.int32),
            pltpu.VMEM((NCHMAX, CH), jnp.int32),
            pltpu.VMEM((CH, H), jnp.bfloat16),
            pltpu.VMEM_SHARED((NPAD, H), jnp.bfloat16),
            pltpu.SemaphoreType.DMA,
        ],
        compiler_params=_SC_PARAMS,
    )(u_bf, srcf, dstf, zeros_bf)


def _dinv_block(deg_ref, base_rows):
    deg = (deg_ref[0][:, 0:1] + deg_ref[1][:, 0:1]).astype(jnp.float32) + 1.0
    rows = base_rows + lax.broadcasted_iota(jnp.int32, deg.shape, 0)
    return jnp.where(rows < N, lax.rsqrt(deg), 0.0)


def _mm_scale_body(deg_ref, x_ref, w_ref, ob_ref):
    i = pl.program_id(0)
    dinv = _dinv_block(deg_ref, i * BM)
    rows = i * BM + lax.broadcasted_iota(jnp.int32, (BM, 1), 0)
    g = jnp.dot(x_ref[...], w_ref[...], preferred_element_type=jnp.float32)
    u = jnp.where(rows < N, g * dinv, 0.0)
    ob_ref[...] = u.astype(jnp.bfloat16)


def _mm_scale(degp, x, W1):
    return pl.pallas_call(
        _mm_scale_body,
        grid=(NPAD // BM,),
        in_specs=[
            pl.BlockSpec((NC, BM, 16), lambda i: (0, i, 0)),
            pl.BlockSpec((BM, D), lambda i: (i, 0)),
            pl.BlockSpec((D, H), lambda i: (0, 0)),
        ],
        out_specs=pl.BlockSpec((BM, H), lambda i: (i, 0)),
        out_shape=jax.ShapeDtypeStruct((NPAD, H), jnp.bfloat16),
    )(degp, x, W1)


def _comb_mm_body(deg_ref, sp_ref, u_ref, b_ref, w_ref, ob_ref):
    i = pl.program_id(0)
    dinv = _dinv_block(deg_ref, i * BM)
    t = (sp_ref[0] + sp_ref[1]).astype(jnp.float32) + u_ref[...].astype(jnp.float32)
    h = jnp.maximum(dinv * t + b_ref[...], 0.0)
    u = dinv * jnp.dot(h, w_ref[...], preferred_element_type=jnp.float32)
    ob_ref[...] = u.astype(jnp.bfloat16)


def _comb_mm(degp, sp, u, b_row, W):
    return pl.pallas_call(
        _comb_mm_body,
        grid=(NPAD // BM,),
        in_specs=[
            pl.BlockSpec((NC, BM, 16), lambda i: (0, i, 0)),
            pl.BlockSpec((NC, BM, H), lambda i: (0, i, 0)),
            pl.BlockSpec((BM, H), lambda i: (i, 0)),
            pl.BlockSpec((1, H), lambda i: (0, 0)),
            pl.BlockSpec((H, H), lambda i: (0, 0)),
        ],
        out_specs=pl.BlockSpec((BM, H), lambda i: (i, 0)),
        out_shape=jax.ShapeDtypeStruct((NPAD, H), jnp.bfloat16),
    )(degp, sp, u, b_row, W)


def _comb_body(deg_ref, sp_ref, u_ref, b_ref, ob_ref):
    i = pl.program_id(0)
    dinv = _dinv_block(deg_ref, i * BM)
    t = (sp_ref[0] + sp_ref[1]).astype(jnp.float32) + u_ref[...].astype(jnp.float32)
    u = dinv * jnp.maximum(dinv * t + b_ref[...], 0.0)
    ob_ref[...] = u.astype(jnp.bfloat16)


def _comb(degp, sp, u, b_row):
    return pl.pallas_call(
        _comb_body,
        grid=(NPAD // BM,),
        in_specs=[
            pl.BlockSpec((NC, BM, 16), lambda i: (0, i, 0)),
            pl.BlockSpec((NC, BM, H), lambda i: (0, i, 0)),
            pl.BlockSpec((BM, H), lambda i: (i, 0)),
            pl.BlockSpec((1, H), lambda i: (0, 0)),
        ],
        out_specs=pl.BlockSpec((BM, H), lambda i: (i, 0)),
        out_shape=jax.ShapeDtypeStruct((NPAD, H), jnp.bfloat16),
    )(degp, sp, u, b_row)


def _final_body(deg_ref, sp_ref, u_ref, wf_ref, bf_ref, o_ref):
    deg = (deg_ref[0][:, 0:1] + deg_ref[1][:, 0:1]).astype(jnp.float32) + 1.0
    dinv = lax.rsqrt(deg)
    t = dinv * ((sp_ref[0] + sp_ref[1]).astype(jnp.float32) + u_ref[...].astype(jnp.float32))
    z = jnp.dot(t, wf_ref[...], preferred_element_type=jnp.float32) + bf_ref[...]
    m = jnp.max(z, axis=1, keepdims=True)
    lse = m + jnp.log(jnp.sum(jnp.exp(z - m), axis=1, keepdims=True))
    o_ref[...] = z - lse


def _final(degp, sp, u, Wf, bf_row):
    return pl.pallas_call(
        _final_body,
        grid=(N // BMF,),
        in_specs=[
            pl.BlockSpec((NC, BMF, 16), lambda i: (0, i, 0)),
            pl.BlockSpec((NC, BMF, H), lambda i: (0, i, 0)),
            pl.BlockSpec((BMF, H), lambda i: (i, 0)),
            pl.BlockSpec((H, C), lambda i: (0, 0)),
            pl.BlockSpec((1, C), lambda i: (0, 0)),
        ],
        out_specs=pl.BlockSpec((BMF, C), lambda i: (i, 0)),
        out_shape=jax.ShapeDtypeStruct((N, C), jnp.float32),
    )(degp, sp, u, Wf, bf_row)


def kernel(x, edge_index, W1, b1, W2, b2, Wf, bf):
    ei = edge_index.astype(jnp.int32)
    tail = jnp.full((CHROWS * CH - E,), PAD_IDX, jnp.int32)
    srcf = jnp.concatenate([ei[0], tail]).reshape(CHROWS, CH)
    dstf = jnp.concatenate([ei[1], tail]).reshape(CHROWS, CH)

    zeros16 = jnp.zeros((NPAD, 16), jnp.bfloat16)
    zeros_bf = jnp.zeros((NPAD, H), jnp.bfloat16)
    ones16 = jnp.ones((CH, 16), jnp.bfloat16)
    b1r = b1.reshape(1, H)
    b2r = b2.reshape(1, H)
    bfr = bf.reshape(1, C)

    degp = _deg_call(dstf, ones16, zeros16)
    u1 = _mm_scale(degp, x, W1)
    s1 = _agg_call(u1, srcf, dstf, zeros_bf)
    u2 = _comb_mm(degp, s1, u1, b1r, W2)
    s2 = _agg_call(u2, srcf, dstf, zeros_bf)
    u3 = _comb(degp, s2, u2, b2r)
    s3 = _agg_call(u3, srcf, dstf, zeros_bf)
    return _final(degp, s3, u3, Wf, bfr)


# speedup vs baseline: 1.0248x; 1.0071x over previous
import jax
import jax.numpy as jnp
from jax import lax
from jax.experimental import pallas as pl
from jax.experimental.pallas import tpu as pltpu
from jax.experimental.pallas import tpu_sc as plsc

N = 10000
E = 320000
D = 128
H = 64
C = 40

NC = 2
NS = 16
CH = 128
TCH = E // CH
T0 = 1250
T1 = TCH - T0
Q0, R0 = T0 // NS, T0 % NS
Q1, R1 = T1 // NS, T1 % NS
NCHMAX = max(Q0, Q1) + 1
CHROWS = TCH + NCHMAX + 8
NPAD = 10240
RPT = NPAD // NS
PAD_IDX = NPAD - 1

BM = 5120
BMF = 2000


def _mesh():
    return plsc.VectorSubcoreMesh(core_axis_name="c", subcore_axis_name="s")


_SC_PARAMS = pltpu.CompilerParams(use_tc_tiling_on_sc=False)


def _tile_range(c, s):
    q = jnp.where(c == 0, Q0, Q1)
    r = jnp.where(c == 0, R0, R1)
    base = jnp.where(c == 0, 0, T0)
    start = base + s * q + jnp.minimum(s, r)
    n = q + (s < r).astype(jnp.int32)
    return start, n


def _deg_body(dstf, ones_hbm, zeros_hbm, out, dst_v, ones_v, acc):
    c = lax.axis_index("c")
    s = lax.axis_index("s")
    start, n = _tile_range(c, s)
    pltpu.sync_copy(zeros_hbm.at[pl.ds(s * RPT, RPT)], acc.at[pl.ds(s * RPT, RPT)])
    pltpu.sync_copy(dstf.at[pl.ds(start, NCHMAX)], dst_v)
    pltpu.sync_copy(ones_hbm, ones_v)
    plsc.subcore_barrier()

    def chunk(i, carry):
        pltpu.sync_copy(ones_v, acc.at[dst_v.at[i]], add=True)
        return carry

    lax.fori_loop(0, n, chunk, 0)
    plsc.subcore_barrier()
    pltpu.sync_copy(acc.at[pl.ds(s * RPT, RPT)], out.at[c, pl.ds(s * RPT, RPT)])


def _deg_call(dstf, ones16, zeros16):
    return pl.kernel(
        _deg_body,
        out_type=jax.ShapeDtypeStruct((NC, NPAD, 16), jnp.bfloat16),
        mesh=_mesh(),
        scratch_types=[
            pltpu.VMEM((NCHMAX, CH), jnp.int32),
            pltpu.VMEM((CH, 16), jnp.bfloat16),
            pltpu.VMEM_SHARED((NPAD, 16), jnp.bfloat16),
        ],
        compiler_params=_SC_PARAMS,
    )(dstf, ones16, zeros16)


def _agg_body(u_hbm, srcf, dstf, zeros_hbm, out, src_v, dst_v, rows_v, acc, sem):
    c = lax.axis_index("c")
    s = lax.axis_index("s")
    start, n = _tile_range(c, s)
    pltpu.sync_copy(zeros_hbm.at[pl.ds(s * RPT, RPT)], acc.at[pl.ds(s * RPT, RPT)])
    pltpu.sync_copy(srcf.at[pl.ds(start, NCHMAX)], src_v)
    pltpu.sync_copy(dstf.at[pl.ds(start, NCHMAX)], dst_v)
    plsc.subcore_barrier()

    def chunk(i, carry):
        pltpu.async_copy(u_hbm.at[src_v.at[i]], rows_v, sem).wait()
        pltpu.sync_copy(rows_v, acc.at[dst_v.at[i]], add=True)
        return carry

    lax.fori_loop(0, n, chunk, 0)
    plsc.subcore_barrier()
    pltpu.sync_copy(acc.at[pl.ds(s * RPT, RPT)], out.at[c, pl.ds(s * RPT, RPT)])


def _agg_call(u_bf, srcf, dstf, zeros_bf):
    return pl.kernel(
        _agg_body,
        out_type=jax.ShapeDtypeStruct((NC, NPAD, H), jnp.bfloat16),
        mesh=_mesh(),
        scratch_types=[
            pltpu.VMEM((NCHMAX, CH), jnp.int32),
            pltpu.VMEM((NCHMAX, CH), jnp.int32),
            pltpu.VMEM((CH, H), jnp.bfloat16),
            pltpu.VMEM_SHARED((NPAD, H), jnp.bfloat16),
            pltpu.SemaphoreType.DMA,
        ],
        compiler_params=_SC_PARAMS,
    )(u_bf, srcf, dstf, zeros_bf)


def _dinv_block(deg_ref, base_rows):
    deg = (deg_ref[0][:, 0:1] + deg_ref[1][:, 0:1]).astype(jnp.float32) + 1.0
    rows = base_rows + lax.broadcasted_iota(jnp.int32, deg.shape, 0)
    return jnp.where(rows < N, lax.rsqrt(deg), 0.0)


def _mm_scale_body(deg_ref, x_ref, w_ref, ob_ref):
    i = pl.program_id(0)
    dinv = _dinv_block(deg_ref, i * BM)
    rows = i * BM + lax.broadcasted_iota(jnp.int32, (BM, 1), 0)
    g = jnp.dot(x_ref[...], w_ref[...], preferred_element_type=jnp.float32)
    u = jnp.where(rows < N, g * dinv, 0.0)
    ob_ref[...] = u.astype(jnp.bfloat16)


def _mm_scale(degp, x, W1):
    return pl.pallas_call(
        _mm_scale_body,
        grid=(NPAD // BM,),
        in_specs=[
            pl.BlockSpec((NC, BM, 16), lambda i: (0, i, 0)),
            pl.BlockSpec((BM, D), lambda i: (i, 0)),
            pl.BlockSpec((D, H), lambda i: (0, 0)),
        ],
        out_specs=pl.BlockSpec((BM, H), lambda i: (i, 0)),
        out_shape=jax.ShapeDtypeStruct((NPAD, H), jnp.bfloat16),
    )(degp, x, W1)


def _comb_mm_body(deg_ref, sp_ref, u_ref, b_ref, w_ref, ob_ref):
    i = pl.program_id(0)
    dinv = _dinv_block(deg_ref, i * BM)
    t = (sp_ref[0] + sp_ref[1]).astype(jnp.float32) + u_ref[...].astype(jnp.float32)
    h = jnp.maximum(dinv * t + b_ref[...], 0.0)
    u = dinv * jnp.dot(h, w_ref[...], preferred_element_type=jnp.float32)
    ob_ref[...] = u.astype(jnp.bfloat16)


def _comb_mm(degp, sp, u, b_row, W):
    return pl.pallas_call(
        _comb_mm_body,
        grid=(NPAD // BM,),
        in_specs=[
            pl.BlockSpec((NC, BM, 16), lambda i: (0, i, 0)),
            pl.BlockSpec((NC, BM, H), lambda i: (0, i, 0)),
            pl.BlockSpec((BM, H), lambda i: (i, 0)),
            pl.BlockSpec((1, H), lambda i: (0, 0)),
            pl.BlockSpec((H, H), lambda i: (0, 0)),
        ],
        out_specs=pl.BlockSpec((BM, H), lambda i: (i, 0)),
        out_shape=jax.ShapeDtypeStruct((NPAD, H), jnp.bfloat16),
    )(degp, sp, u, b_row, W)


def _comb_body(deg_ref, sp_ref, u_ref, b_ref, ob_ref):
    i = pl.program_id(0)
    dinv = _dinv_block(deg_ref, i * BM)
    t = (sp_ref[0] + sp_ref[1]).astype(jnp.float32) + u_ref[...].astype(jnp.float32)
    u = dinv * jnp.maximum(dinv * t + b_ref[...], 0.0)
    ob_ref[...] = u.astype(jnp.bfloat16)


def _comb(degp, sp, u, b_row):
    return pl.pallas_call(
        _comb_body,
        grid=(NPAD // BM,),
        in_specs=[
            pl.BlockSpec((NC, BM, 16), lambda i: (0, i, 0)),
            pl.BlockSpec((NC, BM, H), lambda i: (0, i, 0)),
            pl.BlockSpec((BM, H), lambda i: (i, 0)),
            pl.BlockSpec((1, H), lambda i: (0, 0)),
        ],
        out_specs=pl.BlockSpec((BM, H), lambda i: (i, 0)),
        out_shape=jax.ShapeDtypeStruct((NPAD, H), jnp.bfloat16),
    )(degp, sp, u, b_row)


def _final_body(deg_ref, sp_ref, u_ref, wf_ref, bf_ref, o_ref):
    deg = (deg_ref[0][:, 0:1] + deg_ref[1][:, 0:1]).astype(jnp.float32) + 1.0
    dinv = lax.rsqrt(deg)
    t = dinv * ((sp_ref[0] + sp_ref[1]).astype(jnp.float32) + u_ref[...].astype(jnp.float32))
    z = jnp.dot(t, wf_ref[...], preferred_element_type=jnp.float32) + bf_ref[...]
    m = jnp.max(z, axis=1, keepdims=True)
    lse = m + jnp.log(jnp.sum(jnp.exp(z - m), axis=1, keepdims=True))
    o_ref[...] = z - lse


def _final(degp, sp, u, Wf, bf_row):
    return pl.pallas_call(
        _final_body,
        grid=(N // BMF,),
        in_specs=[
            pl.BlockSpec((NC, BMF, 16), lambda i: (0, i, 0)),
            pl.BlockSpec((NC, BMF, H), lambda i: (0, i, 0)),
            pl.BlockSpec((BMF, H), lambda i: (i, 0)),
            pl.BlockSpec((H, C), lambda i: (0, 0)),
            pl.BlockSpec((1, C), lambda i: (0, 0)),
        ],
        out_specs=pl.BlockSpec((BMF, C), lambda i: (i, 0)),
        out_shape=jax.ShapeDtypeStruct((N, C), jnp.float32),
    )(degp, sp, u, Wf, bf_row)


def kernel(x, edge_index, W1, b1, W2, b2, Wf, bf):
    ei = edge_index.astype(jnp.int32)
    tail = jnp.full((CHROWS * CH - E,), PAD_IDX, jnp.int32)
    srcf = jnp.concatenate([ei[0], tail]).reshape(CHROWS, CH)
    dstf = jnp.concatenate([ei[1], tail]).reshape(CHROWS, CH)

    zeros16 = jnp.zeros((NPAD, 16), jnp.bfloat16)
    zeros_bf = jnp.zeros((NPAD, H), jnp.bfloat16)
    ones16 = jnp.ones((CH, 16), jnp.bfloat16)
    b1r = b1.reshape(1, H)
    b2r = b2.reshape(1, H)
    bfr = bf.reshape(1, C)

    degp = _deg_call(dstf, ones16, zeros16)
    u1 = _mm_scale(degp, x, W1)
    s1 = _agg_call(u1, srcf, dstf, zeros_bf)
    u2 = _comb_mm(degp, s1, u1, b1r, W2)
    s2 = _agg_call(u2, srcf, dstf, zeros_bf)
    u3 = _comb(degp, s2, u2, b2r)
    s3 = _agg_call(u3, srcf, dstf, zeros_bf)
    return _final(degp, s3, u3, Wf, bfr)


# speedup vs baseline: 1.0251x; 1.0003x over previous
import jax
import jax.numpy as jnp
from jax import lax
from jax.experimental import pallas as pl
from jax.experimental.pallas import tpu as pltpu
from jax.experimental.pallas import tpu_sc as plsc

N = 10000
E = 320000
D = 128
H = 64
C = 40

NC = 2
NS = 16
CH = 128
TCH = E // CH
T0 = 1250
T1 = TCH - T0
Q0, R0 = T0 // NS, T0 % NS
Q1, R1 = T1 // NS, T1 % NS
NCHMAX = max(Q0, Q1) + 1
CHROWS = TCH + NCHMAX + 8
NPAD = 10240
RPT = NPAD // NS
PAD_IDX = NPAD - 1

BM = 10240
BMF = 10000


def _mesh():
    return plsc.VectorSubcoreMesh(core_axis_name="c", subcore_axis_name="s")


_SC_PARAMS = pltpu.CompilerParams(use_tc_tiling_on_sc=False)


def _tile_range(c, s):
    q = jnp.where(c == 0, Q0, Q1)
    r = jnp.where(c == 0, R0, R1)
    base = jnp.where(c == 0, 0, T0)
    start = base + s * q + jnp.minimum(s, r)
    n = q + (s < r).astype(jnp.int32)
    return start, n


def _deg_body(dstf, ones_hbm, zeros_hbm, out, dst_v, ones_v, acc):
    c = lax.axis_index("c")
    s = lax.axis_index("s")
    start, n = _tile_range(c, s)
    pltpu.sync_copy(zeros_hbm.at[pl.ds(s * RPT, RPT)], acc.at[pl.ds(s * RPT, RPT)])
    pltpu.sync_copy(dstf.at[pl.ds(start, NCHMAX)], dst_v)
    pltpu.sync_copy(ones_hbm, ones_v)
    plsc.subcore_barrier()

    def chunk(i, carry):
        pltpu.sync_copy(ones_v, acc.at[dst_v.at[i]], add=True)
        return carry

    lax.fori_loop(0, n, chunk, 0)
    plsc.subcore_barrier()
    pltpu.sync_copy(acc.at[pl.ds(s * RPT, RPT)], out.at[c, pl.ds(s * RPT, RPT)])


def _deg_call(dstf, ones16, zeros16):
    return pl.kernel(
        _deg_body,
        out_type=jax.ShapeDtypeStruct((NC, NPAD, 16), jnp.bfloat16),
        mesh=_mesh(),
        scratch_types=[
            pltpu.VMEM((NCHMAX, CH), jnp.int32),
            pltpu.VMEM((CH, 16), jnp.bfloat16),
            pltpu.VMEM_SHARED((NPAD, 16), jnp.bfloat16),
        ],
        compiler_params=_SC_PARAMS,
    )(dstf, ones16, zeros16)


def _agg_body(u_hbm, srcf, dstf, zeros_hbm, out, src_v, dst_v, rows_v, acc, sem):
    c = lax.axis_index("c")
    s = lax.axis_index("s")
    start, n = _tile_range(c, s)
    pltpu.sync_copy(zeros_hbm.at[pl.ds(s * RPT, RPT)], acc.at[pl.ds(s * RPT, RPT)])
    pltpu.sync_copy(srcf.at[pl.ds(start, NCHMAX)], src_v)
    pltpu.sync_copy(dstf.at[pl.ds(start, NCHMAX)], dst_v)
    plsc.subcore_barrier()

    def chunk(i, carry):
        pltpu.async_copy(u_hbm.at[src_v.at[i]], rows_v, sem).wait()
        pltpu.sync_copy(rows_v, acc.at[dst_v.at[i]], add=True)
        return carry

    lax.fori_loop(0, n, chunk, 0)
    plsc.subcore_barrier()
    pltpu.sync_copy(acc.at[pl.ds(s * RPT, RPT)], out.at[c, pl.ds(s * RPT, RPT)])


def _agg_call(u_bf, srcf, dstf, zeros_bf):
    return pl.kernel(
        _agg_body,
        out_type=jax.ShapeDtypeStruct((NC, NPAD, H), jnp.bfloat16),
        mesh=_mesh(),
        scratch_types=[
            pltpu.VMEM((NCHMAX, CH), jnp.int32),
            pltpu.VMEM((NCHMAX, CH), jnp.int32),
            pltpu.VMEM((CH, H), jnp.bfloat16),
            pltpu.VMEM_SHARED((NPAD, H), jnp.bfloat16),
            pltpu.SemaphoreType.DMA,
        ],
        compiler_params=_SC_PARAMS,
    )(u_bf, srcf, dstf, zeros_bf)


def _dinv_block(deg_ref, base_rows):
    deg = (deg_ref[0][:, 0:1] + deg_ref[1][:, 0:1]).astype(jnp.float32) + 1.0
    rows = base_rows + lax.broadcasted_iota(jnp.int32, deg.shape, 0)
    return jnp.where(rows < N, lax.rsqrt(deg), 0.0)


def _mm_scale_body(deg_ref, x_ref, w_ref, ob_ref):
    i = pl.program_id(0)
    dinv = _dinv_block(deg_ref, i * BM)
    rows = i * BM + lax.broadcasted_iota(jnp.int32, (BM, 1), 0)
    g = jnp.dot(x_ref[...], w_ref[...], preferred_element_type=jnp.float32)
    u = jnp.where(rows < N, g * dinv, 0.0)
    ob_ref[...] = u.astype(jnp.bfloat16)


def _mm_scale(degp, x, W1):
    return pl.pallas_call(
        _mm_scale_body,
        grid=(NPAD // BM,),
        in_specs=[
            pl.BlockSpec((NC, BM, 16), lambda i: (0, i, 0)),
            pl.BlockSpec((BM, D), lambda i: (i, 0)),
            pl.BlockSpec((D, H), lambda i: (0, 0)),
        ],
        out_specs=pl.BlockSpec((BM, H), lambda i: (i, 0)),
        out_shape=jax.ShapeDtypeStruct((NPAD, H), jnp.bfloat16),
    )(degp, x, W1)


def _comb_mm_body(deg_ref, sp_ref, u_ref, b_ref, w_ref, ob_ref):
    i = pl.program_id(0)
    dinv = _dinv_block(deg_ref, i * BM)
    t = (sp_ref[0] + sp_ref[1]).astype(jnp.float32) + u_ref[...].astype(jnp.float32)
    h = jnp.maximum(dinv * t + b_ref[...], 0.0)
    u = dinv * jnp.dot(h, w_ref[...], preferred_element_type=jnp.float32)
    ob_ref[...] = u.astype(jnp.bfloat16)


def _comb_mm(degp, sp, u, b_row, W):
    return pl.pallas_call(
        _comb_mm_body,
        grid=(NPAD // BM,),
        in_specs=[
            pl.BlockSpec((NC, BM, 16), lambda i: (0, i, 0)),
            pl.BlockSpec((NC, BM, H), lambda i: (0, i, 0)),
            pl.BlockSpec((BM, H), lambda i: (i, 0)),
            pl.BlockSpec((1, H), lambda i: (0, 0)),
            pl.BlockSpec((H, H), lambda i: (0, 0)),
        ],
        out_specs=pl.BlockSpec((BM, H), lambda i: (i, 0)),
        out_shape=jax.ShapeDtypeStruct((NPAD, H), jnp.bfloat16),
    )(degp, sp, u, b_row, W)


def _comb_body(deg_ref, sp_ref, u_ref, b_ref, ob_ref):
    i = pl.program_id(0)
    dinv = _dinv_block(deg_ref, i * BM)
    t = (sp_ref[0] + sp_ref[1]).astype(jnp.float32) + u_ref[...].astype(jnp.float32)
    u = dinv * jnp.maximum(dinv * t + b_ref[...], 0.0)
    ob_ref[...] = u.astype(jnp.bfloat16)


def _comb(degp, sp, u, b_row):
    return pl.pallas_call(
        _comb_body,
        grid=(NPAD // BM,),
        in_specs=[
            pl.BlockSpec((NC, BM, 16), lambda i: (0, i, 0)),
            pl.BlockSpec((NC, BM, H), lambda i: (0, i, 0)),
            pl.BlockSpec((BM, H), lambda i: (i, 0)),
            pl.BlockSpec((1, H), lambda i: (0, 0)),
        ],
        out_specs=pl.BlockSpec((BM, H), lambda i: (i, 0)),
        out_shape=jax.ShapeDtypeStruct((NPAD, H), jnp.bfloat16),
    )(degp, sp, u, b_row)


def _final_body(deg_ref, sp_ref, u_ref, wf_ref, bf_ref, o_ref):
    deg = (deg_ref[0][:, 0:1] + deg_ref[1][:, 0:1]).astype(jnp.float32) + 1.0
    dinv = lax.rsqrt(deg)
    t = dinv * ((sp_ref[0] + sp_ref[1]).astype(jnp.float32) + u_ref[...].astype(jnp.float32))
    z = jnp.dot(t, wf_ref[...], preferred_element_type=jnp.float32) + bf_ref[...]
    m = jnp.max(z, axis=1, keepdims=True)
    lse = m + jnp.log(jnp.sum(jnp.exp(z - m), axis=1, keepdims=True))
    o_ref[...] = z - lse


def _final(degp, sp, u, Wf, bf_row):
    return pl.pallas_call(
        _final_body,
        grid=(N // BMF,),
        in_specs=[
            pl.BlockSpec((NC, BMF, 16), lambda i: (0, i, 0)),
            pl.BlockSpec((NC, BMF, H), lambda i: (0, i, 0)),
            pl.BlockSpec((BMF, H), lambda i: (i, 0)),
            pl.BlockSpec((H, C), lambda i: (0, 0)),
            pl.BlockSpec((1, C), lambda i: (0, 0)),
        ],
        out_specs=pl.BlockSpec((BMF, C), lambda i: (i, 0)),
        out_shape=jax.ShapeDtypeStruct((N, C), jnp.float32),
    )(degp, sp, u, Wf, bf_row)


def kernel(x, edge_index, W1, b1, W2, b2, Wf, bf):
    ei = edge_index.astype(jnp.int32)
    tail = jnp.full((CHROWS * CH - E,), PAD_IDX, jnp.int32)
    srcf = jnp.concatenate([ei[0], tail]).reshape(CHROWS, CH)
    dstf = jnp.concatenate([ei[1], tail]).reshape(CHROWS, CH)

    zeros16 = jnp.zeros((NPAD, 16), jnp.bfloat16)
    zeros_bf = jnp.zeros((NPAD, H), jnp.bfloat16)
    ones16 = jnp.ones((CH, 16), jnp.bfloat16)
    b1r = b1.reshape(1, H)
    b2r = b2.reshape(1, H)
    bfr = bf.reshape(1, C)

    degp = _deg_call(dstf, ones16, zeros16)
    u1 = _mm_scale(degp, x, W1)
    s1 = _agg_call(u1, srcf, dstf, zeros_bf)
    u2 = _comb_mm(degp, s1, u1, b1r, W2)
    s2 = _agg_call(u2, srcf, dstf, zeros_bf)
    u3 = _comb(degp, s2, u2, b2r)
    s3 = _agg_call(u3, srcf, dstf, zeros_bf)
    return _final(degp, s3, u3, Wf, bfr)
